# Initial kernel scaffold; baseline (speedup 1.0000x reference)
import jax
import jax.numpy as jnp
from jax.experimental import pallas as pl


def kernel(x, neighbor_idx, W_self1, W_neigh1, b1, Wih1, Whh1, bih1, bhh1, W_self2, W_neigh2, b2, Wih2, Whh2, bih2, bhh2, fc1_W, fc1_b, fc3_W, fc3_b):
    raise NotImplementedError("write your pallas kernel here")


# speedup vs baseline: 3.7177x; 3.7177x over previous
import functools

import jax
import jax.numpy as jnp
from jax import lax
from jax.experimental import pallas as pl
from jax.experimental.pallas import tpu as pltpu
from jax.experimental.pallas import tpu_sc as plsc

N = 10000
DEG = 32
D = 128
H = 128

NC = 2
NS = 16
NW = NC * NS
R = N * DEG
PW = R // NW
CH = 400
NCH = PW // CH


def _sc_gather(table, flat_idx):
    mesh = plsc.VectorSubcoreMesh(core_axis_name="c", subcore_axis_name="s")

    @functools.partial(
        pl.kernel,
        mesh=mesh,
        out_type=jax.ShapeDtypeStruct((R, D), jnp.float32),
        scratch_types=[
            pltpu.VMEM((PW,), jnp.int32),
            pltpu.VMEM((CH, D), jnp.float32),
            pltpu.VMEM((CH, D), jnp.float32),
            pltpu.SemaphoreType.DMA,
            pltpu.SemaphoreType.DMA,
        ],
    )
    def k(table_hbm, idx_hbm, out_hbm, idx_v, buf0, buf1, sem0, sem1):
        wid = lax.axis_index("s") * NC + lax.axis_index("c")
        base = wid * PW
        pltpu.sync_copy(idx_hbm.at[pl.ds(base, PW)], idx_v)
        bufs = (buf0, buf1)
        sems = (sem0, sem1)
        handles = [None] * NCH
        handles[0] = pltpu.async_copy(
            table_hbm.at[idx_v.at[pl.ds(0, CH)]], bufs[0], sems[0])
        for j in range(NCH):
            if j + 1 < NCH:
                handles[j + 1] = pltpu.async_copy(
                    table_hbm.at[idx_v.at[pl.ds((j + 1) * CH, CH)]],
                    bufs[(j + 1) % 2], sems[(j + 1) % 2])
            handles[j].wait()
            pltpu.sync_copy(bufs[j % 2], out_hbm.at[pl.ds(base + j * CH, CH)])

    return k(table, flat_idx)


def _lstm_tile(m_ref, wih, whh, bias_g, nrows):
    h0 = jnp.zeros((nrows, H), jnp.float32)
    c0 = jnp.zeros((nrows, H), jnp.float32)

    def step(t, hc):
        h, c = hc
        xt = m_ref[t]
        gates = (jnp.dot(xt, wih, preferred_element_type=jnp.float32)
                 + jnp.dot(h, whh, preferred_element_type=jnp.float32)
                 + bias_g)
        i = jax.nn.sigmoid(gates[:, :H])
        f = jax.nn.sigmoid(gates[:, H:2 * H])
        g = jnp.tanh(gates[:, 2 * H:3 * H])
        o = jax.nn.sigmoid(gates[:, 3 * H:])
        c2 = f * c + i * g
        h2 = o * jnp.tanh(c2)
        return (h2, c2)

    h, _ = lax.fori_loop(0, DEG, step, (h0, c0))
    return h


B1 = 1000


def _layer1_call(xin, mg, Wih, Whh, bias_g, Ws, Wn, bias_o):
    nt = N // B1

    def body(x_ref, m_ref, wih_ref, whh_ref, bg_ref, ws_ref, wn_ref, bo_ref,
             o_ref):
        h = _lstm_tile(m_ref, wih_ref[...], whh_ref[...], bg_ref[...], B1)
        o_ref[...] = (
            jnp.dot(x_ref[...], ws_ref[...], preferred_element_type=jnp.float32)
            + jnp.dot(h, wn_ref[...], preferred_element_type=jnp.float32)
            + bo_ref[...])

    return pl.pallas_call(
        body,
        grid=(nt,),
        in_specs=[
            pl.BlockSpec((B1, D), lambda i: (i, 0)),
            pl.BlockSpec((DEG, B1, D), lambda i: (0, i, 0)),
            pl.BlockSpec((D, 4 * H), lambda i: (0, 0)),
            pl.BlockSpec((H, 4 * H), lambda i: (0, 0)),
            pl.BlockSpec((1, 4 * H), lambda i: (0, 0)),
            pl.BlockSpec((D, H), lambda i: (0, 0)),
            pl.BlockSpec((H, H), lambda i: (0, 0)),
            pl.BlockSpec((1, H), lambda i: (0, 0)),
        ],
        out_specs=pl.BlockSpec((B1, H), lambda i: (i, 0)),
        out_shape=jax.ShapeDtypeStruct((N, H), jnp.float32),
    )(xin, mg, Wih, Whh, bias_g, Ws, Wn, bias_o)


def _layer2_head_call(xin, mg, Wih, Whh, bias_g, Ws, Wn, bias_o,
                      fc1_W, fc1_b, fc3_W, fc3_b):
    nt = N // B1

    def body(x_ref, m_ref, wih_ref, whh_ref, bg_ref, ws_ref, wn_ref, bo_ref,
             fc1w_ref, fc1b_ref, fc3w_ref, fc3b_ref, o_ref, mx_ref):
        i = pl.program_id(0)
        h = _lstm_tile(m_ref, wih_ref[...], whh_ref[...], bg_ref[...], B1)
        h2 = (jnp.dot(x_ref[...], ws_ref[...],
                      preferred_element_type=jnp.float32)
              + jnp.dot(h, wn_ref[...], preferred_element_type=jnp.float32)
              + bo_ref[...])
        tmax = jnp.max(h2, axis=0, keepdims=True)

        @pl.when(i == 0)
        def _():
            mx_ref[...] = tmax

        @pl.when(i > 0)
        def _():
            mx_ref[...] = jnp.maximum(mx_ref[...], tmax)

        @pl.when(i == nt - 1)
        def _():
            g = mx_ref[...]
            z = jnp.maximum(
                jnp.dot(g, fc1w_ref[...], preferred_element_type=jnp.float32)
                + fc1b_ref[...], 0.0)
            o_ref[...] = (jnp.dot(z, fc3w_ref[...],
                                  preferred_element_type=jnp.float32)
                          + fc3b_ref[...])

    return pl.pallas_call(
        body,
        grid=(nt,),
        in_specs=[
            pl.BlockSpec((B1, H), lambda i: (i, 0)),
            pl.BlockSpec((DEG, B1, H), lambda i: (0, i, 0)),
            pl.BlockSpec((H, 4 * H), lambda i: (0, 0)),
            pl.BlockSpec((H, 4 * H), lambda i: (0, 0)),
            pl.BlockSpec((1, 4 * H), lambda i: (0, 0)),
            pl.BlockSpec((H, H), lambda i: (0, 0)),
            pl.BlockSpec((H, H), lambda i: (0, 0)),
            pl.BlockSpec((1, H), lambda i: (0, 0)),
            pl.BlockSpec((H, H // 2), lambda i: (0, 0)),
            pl.BlockSpec((1, H // 2), lambda i: (0, 0)),
            pl.BlockSpec((H // 2, 1), lambda i: (0, 0)),
            pl.BlockSpec((1, 1), lambda i: (0, 0)),
        ],
        out_specs=pl.BlockSpec((1, 1), lambda i: (0, 0)),
        out_shape=jax.ShapeDtypeStruct((1, 1), jnp.float32),
        scratch_shapes=[pltpu.VMEM((1, H), jnp.float32)],
    )(xin, mg, Wih, Whh, bias_g, Ws, Wn, bias_o, fc1_W, fc1_b, fc3_W, fc3_b)


def kernel(x, neighbor_idx, W_self1, W_neigh1, b1, Wih1, Whh1, bih1, bhh1,
           W_self2, W_neigh2, b2, Wih2, Whh2, bih2, bhh2,
           fc1_W, fc1_b, fc3_W, fc3_b):
    flat_idx = neighbor_idx.astype(jnp.int32).T.reshape(-1)

    bg1 = (bih1 + bhh1).reshape(1, 4 * H)
    bg2 = (bih2 + bhh2).reshape(1, 4 * H)
    bo1 = b1.reshape(1, H)
    bo2 = b2.reshape(1, H)

    mg1 = _sc_gather(x, flat_idx).reshape(DEG, N, D)
    h1 = _layer1_call(x, mg1, Wih1, Whh1, bg1, W_self1, W_neigh1, bo1)

    mg2 = _sc_gather(h1, flat_idx).reshape(DEG, N, H)
    out = _layer2_head_call(h1, mg2, Wih2, Whh2, bg2, W_self2, W_neigh2, bo2,
                            fc1_W, fc1_b.reshape(1, H // 2),
                            fc3_W, fc3_b.reshape(1, 1))
    return out


# speedup vs baseline: 4.3411x; 1.1677x over previous
import functools

import jax
import jax.numpy as jnp
from jax import lax
from jax.experimental import pallas as pl
from jax.experimental.pallas import tpu as pltpu
from jax.experimental.pallas import tpu_sc as plsc

N = 10000
DEG = 32
D = 128
H = 128

NC = 2
NS = 16
NW = NC * NS
R = N * DEG
PW = R // NW
CH = 400
NCH = PW // CH


def _sc_gather(table, flat_idx):
    mesh = plsc.VectorSubcoreMesh(core_axis_name="c", subcore_axis_name="s")
    dt = table.dtype

    @functools.partial(
        pl.kernel,
        mesh=mesh,
        out_type=jax.ShapeDtypeStruct((R, D), dt),
        scratch_types=[
            pltpu.VMEM((PW,), jnp.int32),
            pltpu.VMEM((CH, D), dt),
            pltpu.VMEM((CH, D), dt),
            pltpu.SemaphoreType.DMA,
            pltpu.SemaphoreType.DMA,
        ],
    )
    def k(table_hbm, idx_hbm, out_hbm, idx_v, buf0, buf1, sem0, sem1):
        wid = lax.axis_index("s") * NC + lax.axis_index("c")
        base = wid * PW
        pltpu.sync_copy(idx_hbm.at[pl.ds(base, PW)], idx_v)
        bufs = (buf0, buf1)
        sems = (sem0, sem1)
        handles = [None] * NCH
        handles[0] = pltpu.async_copy(
            table_hbm.at[idx_v.at[pl.ds(0, CH)]], bufs[0], sems[0])
        for j in range(NCH):
            if j + 1 < NCH:
                handles[j + 1] = pltpu.async_copy(
                    table_hbm.at[idx_v.at[pl.ds((j + 1) * CH, CH)]],
                    bufs[(j + 1) % 2], sems[(j + 1) % 2])
            handles[j].wait()
            pltpu.sync_copy(bufs[j % 2], out_hbm.at[pl.ds(base + j * CH, CH)])

    return k(table, flat_idx)


def _lstm_tile(m_ref, w_cat, bias_g, nrows):
    h0 = jnp.zeros((nrows, H), jnp.float32)
    c0 = jnp.zeros((nrows, H), jnp.float32)

    def step(t, hc):
        h, c = hc
        xt = m_ref[t].astype(jnp.bfloat16)
        xh = jnp.concatenate([xt, h.astype(jnp.bfloat16)], axis=1)
        gates = jnp.dot(xh, w_cat, preferred_element_type=jnp.float32) + bias_g
        i = jax.nn.sigmoid(gates[:, :H])
        f = jax.nn.sigmoid(gates[:, H:2 * H])
        g = jnp.tanh(gates[:, 2 * H:3 * H])
        o = jax.nn.sigmoid(gates[:, 3 * H:])
        c2 = f * c + i * g
        h2 = o * jnp.tanh(c2)
        return (h2, c2)

    h, _ = lax.fori_loop(0, DEG, step, (h0, c0))
    return h


B1 = 1000


def _layer1_call(xin, mg, w_cat, bias_g, w_sn, bias_o):
    nt = N // B1

    def body(x_ref, m_ref, wc_ref, bg_ref, wsn_ref, bo_ref, o_ref):
        h = _lstm_tile(m_ref, wc_ref[...], bg_ref[...], B1)
        xh = jnp.concatenate([x_ref[...], h], axis=1)
        o_ref[...] = (jnp.dot(xh, wsn_ref[...],
                              preferred_element_type=jnp.float32)
                      + bo_ref[...])

    return pl.pallas_call(
        body,
        grid=(nt,),
        in_specs=[
            pl.BlockSpec((B1, D), lambda i: (i, 0)),
            pl.BlockSpec((DEG, B1, D), lambda i: (0, i, 0)),
            pl.BlockSpec((2 * D, 4 * H), lambda i: (0, 0)),
            pl.BlockSpec((1, 4 * H), lambda i: (0, 0)),
            pl.BlockSpec((2 * D, H), lambda i: (0, 0)),
            pl.BlockSpec((1, H), lambda i: (0, 0)),
        ],
        out_specs=pl.BlockSpec((B1, H), lambda i: (i, 0)),
        out_shape=jax.ShapeDtypeStruct((N, H), jnp.float32),
    )(xin, mg, w_cat, bias_g, w_sn, bias_o)


def _layer2_head_call(xin, mg, w_cat, bias_g, w_sn, bias_o,
                      fc1_W, fc1_b, fc3_W, fc3_b):
    nt = N // B1

    def body(x_ref, m_ref, wc_ref, bg_ref, wsn_ref, bo_ref,
             fc1w_ref, fc1b_ref, fc3w_ref, fc3b_ref, o_ref, mx_ref):
        i = pl.program_id(0)
        h = _lstm_tile(m_ref, wc_ref[...], bg_ref[...], B1)
        xh = jnp.concatenate([x_ref[...], h], axis=1)
        h2 = (jnp.dot(xh, wsn_ref[...], preferred_element_type=jnp.float32)
              + bo_ref[...])
        tmax = jnp.max(h2, axis=0, keepdims=True)

        @pl.when(i == 0)
        def _():
            mx_ref[...] = tmax

        @pl.when(i > 0)
        def _():
            mx_ref[...] = jnp.maximum(mx_ref[...], tmax)

        @pl.when(i == nt - 1)
        def _():
            g = mx_ref[...]
            z = jnp.maximum(
                jnp.dot(g, fc1w_ref[...], preferred_element_type=jnp.float32)
                + fc1b_ref[...], 0.0)
            o_ref[...] = (jnp.dot(z, fc3w_ref[...],
                                  preferred_element_type=jnp.float32)
                          + fc3b_ref[...])

    return pl.pallas_call(
        body,
        grid=(nt,),
        in_specs=[
            pl.BlockSpec((B1, H), lambda i: (i, 0)),
            pl.BlockSpec((DEG, B1, H), lambda i: (0, i, 0)),
            pl.BlockSpec((2 * H, 4 * H), lambda i: (0, 0)),
            pl.BlockSpec((1, 4 * H), lambda i: (0, 0)),
            pl.BlockSpec((2 * H, H), lambda i: (0, 0)),
            pl.BlockSpec((1, H), lambda i: (0, 0)),
            pl.BlockSpec((H, H // 2), lambda i: (0, 0)),
            pl.BlockSpec((1, H // 2), lambda i: (0, 0)),
            pl.BlockSpec((H // 2, 1), lambda i: (0, 0)),
            pl.BlockSpec((1, 1), lambda i: (0, 0)),
        ],
        out_specs=pl.BlockSpec((1, 1), lambda i: (0, 0)),
        out_shape=jax.ShapeDtypeStruct((1, 1), jnp.float32),
        scratch_shapes=[pltpu.VMEM((1, H), jnp.float32)],
    )(xin, mg, w_cat, bias_g, w_sn, bias_o, fc1_W, fc1_b, fc3_W, fc3_b)


def kernel(x, neighbor_idx, W_self1, W_neigh1, b1, Wih1, Whh1, bih1, bhh1,
           W_self2, W_neigh2, b2, Wih2, Whh2, bih2, bhh2,
           fc1_W, fc1_b, fc3_W, fc3_b):
    bf = jnp.bfloat16
    flat_idx = neighbor_idx.astype(jnp.int32).T.reshape(-1)

    wc1 = jnp.concatenate([Wih1, Whh1], axis=0).astype(bf)
    wc2 = jnp.concatenate([Wih2, Whh2], axis=0).astype(bf)
    wsn1 = jnp.concatenate([W_self1, W_neigh1], axis=0)
    wsn2 = jnp.concatenate([W_self2, W_neigh2], axis=0)
    bg1 = (bih1 + bhh1).reshape(1, 4 * H)
    bg2 = (bih2 + bhh2).reshape(1, 4 * H)
    bo1 = b1.reshape(1, H)
    bo2 = b2.reshape(1, H)

    mg1 = _sc_gather(x, flat_idx).reshape(DEG, N, D)
    h1 = _layer1_call(x, mg1, wc1, bg1, wsn1, bo1)

    mg2 = _sc_gather(h1, flat_idx).reshape(DEG, N, H)
    out = _layer2_head_call(h1, mg2, wc2, bg2, wsn2, bo2,
                            fc1_W, fc1_b.reshape(1, H // 2),
                            fc3_W, fc3_b.reshape(1, 1))
    return out


# speedup vs baseline: 4.6240x; 1.0651x over previous
import functools

import jax
import jax.numpy as jnp
from jax import lax
from jax.experimental import pallas as pl
from jax.experimental.pallas import tpu as pltpu
from jax.experimental.pallas import tpu_sc as plsc

N = 10000
DEG = 32
D = 128
H = 128

NC = 2
NS = 16
NW = NC * NS
R = N * DEG
PW = R // NW
CH = 400
NCH = PW // CH


def _sigmoid(x):
    return jnp.tanh(x * 0.5) * 0.5 + 0.5


def _sc_gather(table, flat_idx):
    mesh = plsc.VectorSubcoreMesh(core_axis_name="c", subcore_axis_name="s")

    @functools.partial(
        pl.kernel,
        mesh=mesh,
        out_type=jax.ShapeDtypeStruct((R, D), jnp.float32),
        scratch_types=[
            pltpu.VMEM((PW,), jnp.int32),
            pltpu.VMEM((CH, D), jnp.float32),
            pltpu.VMEM((CH, D), jnp.float32),
            pltpu.SemaphoreType.DMA,
            pltpu.SemaphoreType.DMA,
        ],
    )
    def k(table_hbm, idx_hbm, out_hbm, idx_v, buf0, buf1, sem0, sem1):
        wid = lax.axis_index("s") * NC + lax.axis_index("c")
        base = wid * PW
        pltpu.sync_copy(idx_hbm.at[pl.ds(base, PW)], idx_v)
        bufs = (buf0, buf1)
        sems = (sem0, sem1)
        handles = [None] * NCH
        handles[0] = pltpu.async_copy(
            table_hbm.at[idx_v.at[pl.ds(0, CH)]], bufs[0], sems[0])
        for j in range(NCH):
            if j + 1 < NCH:
                handles[j + 1] = pltpu.async_copy(
                    table_hbm.at[idx_v.at[pl.ds((j + 1) * CH, CH)]],
                    bufs[(j + 1) % 2], sems[(j + 1) % 2])
            handles[j].wait()
            pltpu.sync_copy(bufs[j % 2], out_hbm.at[pl.ds(base + j * CH, CH)])

    return k(table, flat_idx)


def _lstm_tile(m_ref, w_cat, bias_g, nrows):
    h0 = jnp.zeros((nrows, H), jnp.float32)
    c0 = jnp.zeros((nrows, H), jnp.float32)

    def step(t, hc):
        h, c = hc
        xt = m_ref[t].astype(jnp.bfloat16)
        xh = jnp.concatenate([xt, h.astype(jnp.bfloat16)], axis=1)
        gates = jnp.dot(xh, w_cat, preferred_element_type=jnp.float32) + bias_g
        i = _sigmoid(gates[:, :H])
        f = _sigmoid(gates[:, H:2 * H])
        g = jnp.tanh(gates[:, 2 * H:3 * H])
        o = _sigmoid(gates[:, 3 * H:])
        c2 = f * c + i * g
        h2 = o * jnp.tanh(c2)
        return (h2, c2)

    h, _ = lax.fori_loop(0, DEG, step, (h0, c0))
    return h


B1 = 1000


def _layer1_call(xin, mg, w_cat, bias_g, w_sn, bias_o):
    nt = N // B1

    def body(x_ref, m_ref, wc_ref, bg_ref, wsn_ref, bo_ref, o_ref):
        h = _lstm_tile(m_ref, wc_ref[...], bg_ref[...], B1)
        xh = jnp.concatenate([x_ref[...], h], axis=1)
        o_ref[...] = (jnp.dot(xh, wsn_ref[...],
                              preferred_element_type=jnp.float32)
                      + bo_ref[...])

    return pl.pallas_call(
        body,
        grid=(nt,),
        in_specs=[
            pl.BlockSpec((B1, D), lambda i: (i, 0)),
            pl.BlockSpec((DEG, B1, D), lambda i: (0, i, 0)),
            pl.BlockSpec((2 * D, 4 * H), lambda i: (0, 0)),
            pl.BlockSpec((1, 4 * H), lambda i: (0, 0)),
            pl.BlockSpec((2 * D, H), lambda i: (0, 0)),
            pl.BlockSpec((1, H), lambda i: (0, 0)),
        ],
        out_specs=pl.BlockSpec((B1, H), lambda i: (i, 0)),
        out_shape=jax.ShapeDtypeStruct((N, H), jnp.float32),
    )(xin, mg, w_cat, bias_g, w_sn, bias_o)


def _layer2_head_call(xin, mg, w_cat, bias_g, w_sn, bias_o,
                      fc1_W, fc1_b, fc3_W, fc3_b):
    nt = N // B1

    def body(x_ref, m_ref, wc_ref, bg_ref, wsn_ref, bo_ref,
             fc1w_ref, fc1b_ref, fc3w_ref, fc3b_ref, o_ref, mx_ref):
        i = pl.program_id(0)
        h = _lstm_tile(m_ref, wc_ref[...], bg_ref[...], B1)
        xh = jnp.concatenate([x_ref[...], h], axis=1)
        h2 = (jnp.dot(xh, wsn_ref[...], preferred_element_type=jnp.float32)
              + bo_ref[...])
        tmax = jnp.max(h2, axis=0, keepdims=True)

        @pl.when(i == 0)
        def _():
            mx_ref[...] = tmax

        @pl.when(i > 0)
        def _():
            mx_ref[...] = jnp.maximum(mx_ref[...], tmax)

        @pl.when(i == nt - 1)
        def _():
            g = mx_ref[...]
            z = jnp.maximum(
                jnp.dot(g, fc1w_ref[...], preferred_element_type=jnp.float32)
                + fc1b_ref[...], 0.0)
            o_ref[...] = (jnp.dot(z, fc3w_ref[...],
                                  preferred_element_type=jnp.float32)
                          + fc3b_ref[...])

    return pl.pallas_call(
        body,
        grid=(nt,),
        in_specs=[
            pl.BlockSpec((B1, H), lambda i: (i, 0)),
            pl.BlockSpec((DEG, B1, D), lambda i: (0, i, 0)),
            pl.BlockSpec((2 * H, 4 * H), lambda i: (0, 0)),
            pl.BlockSpec((1, 4 * H), lambda i: (0, 0)),
            pl.BlockSpec((2 * H, H), lambda i: (0, 0)),
            pl.BlockSpec((1, H), lambda i: (0, 0)),
            pl.BlockSpec((H, H // 2), lambda i: (0, 0)),
            pl.BlockSpec((1, H // 2), lambda i: (0, 0)),
            pl.BlockSpec((H // 2, 1), lambda i: (0, 0)),
            pl.BlockSpec((1, 1), lambda i: (0, 0)),
        ],
        out_specs=pl.BlockSpec((1, 1), lambda i: (0, 0)),
        out_shape=jax.ShapeDtypeStruct((1, 1), jnp.float32),
        scratch_shapes=[pltpu.VMEM((1, H), jnp.float32)],
    )(xin, mg, w_cat, bias_g, w_sn, bias_o, fc1_W, fc1_b, fc3_W, fc3_b)


def kernel(x, neighbor_idx, W_self1, W_neigh1, b1, Wih1, Whh1, bih1, bhh1,
           W_self2, W_neigh2, b2, Wih2, Whh2, bih2, bhh2,
           fc1_W, fc1_b, fc3_W, fc3_b):
    bf = jnp.bfloat16
    flat_idx = neighbor_idx.astype(jnp.int32).T.reshape(-1)

    wc1 = jnp.concatenate([Wih1, Whh1], axis=0).astype(bf)
    wc2 = jnp.concatenate([Wih2, Whh2], axis=0).astype(bf)
    wsn1 = jnp.concatenate([W_self1, W_neigh1], axis=0)
    wsn2 = jnp.concatenate([W_self2, W_neigh2], axis=0)
    bg1 = (bih1 + bhh1).reshape(1, 4 * H)
    bg2 = (bih2 + bhh2).reshape(1, 4 * H)
    bo1 = b1.reshape(1, H)
    bo2 = b2.reshape(1, H)

    mg1 = _sc_gather(x, flat_idx).reshape(DEG, N, D)
    h1 = _layer1_call(x, mg1, wc1, bg1, wsn1, bo1)

    mg2 = _sc_gather(h1, flat_idx).reshape(DEG, N, H)
    out = _layer2_head_call(h1, mg2, wc2, bg2, wsn2, bo2,
                            fc1_W, fc1_b.reshape(1, H // 2),
                            fc3_W, fc3_b.reshape(1, 1))
    return out


# speedup vs baseline: 4.8635x; 1.0518x over previous
import functools

import jax
import jax.numpy as jnp
from jax import lax
from jax.experimental import pallas as pl
from jax.experimental.pallas import tpu as pltpu
from jax.experimental.pallas import tpu_sc as plsc

N = 10000
DEG = 32
D = 128
H = 128

NC = 2
NS = 16
NW = NC * NS
R = N * DEG
PW = R // NW
CH = 400
NCH = PW // CH


def _sigmoid_half(x):
    return jnp.tanh(x) * 0.5 + 0.5


def _fold_half(w):
    return jnp.concatenate(
        [w[:, :2 * H] * 0.5, w[:, 2 * H:3 * H], w[:, 3 * H:] * 0.5], axis=1)


def _sc_gather(table, flat_idx):
    mesh = plsc.VectorSubcoreMesh(core_axis_name="c", subcore_axis_name="s")

    @functools.partial(
        pl.kernel,
        mesh=mesh,
        out_type=jax.ShapeDtypeStruct((R, D), jnp.float32),
        scratch_types=[
            pltpu.VMEM((PW,), jnp.int32),
            pltpu.VMEM((CH, D), jnp.float32),
            pltpu.VMEM((CH, D), jnp.float32),
            pltpu.SemaphoreType.DMA,
            pltpu.SemaphoreType.DMA,
            pltpu.SemaphoreType.DMA,
            pltpu.SemaphoreType.DMA,
        ],
    )
    def k(table_hbm, idx_hbm, out_hbm, idx_v, buf0, buf1,
          gsem0, gsem1, ssem0, ssem1):
        wid = lax.axis_index("s") * NC + lax.axis_index("c")
        base = wid * PW
        pltpu.sync_copy(idx_hbm.at[pl.ds(base, PW)], idx_v)
        bufs = (buf0, buf1)
        gsems = (gsem0, gsem1)
        ssems = (ssem0, ssem1)
        gh = [None] * NCH
        sh = [None] * NCH
        gh[0] = pltpu.async_copy(
            table_hbm.at[idx_v.at[pl.ds(0, CH)]], bufs[0], gsems[0])
        for j in range(NCH):
            if j + 1 < NCH:
                if j >= 1:
                    sh[j - 1].wait()
                gh[j + 1] = pltpu.async_copy(
                    table_hbm.at[idx_v.at[pl.ds((j + 1) * CH, CH)]],
                    bufs[(j + 1) % 2], gsems[(j + 1) % 2])
            gh[j].wait()
            sh[j] = pltpu.async_copy(
                bufs[j % 2], out_hbm.at[pl.ds(base + j * CH, CH)],
                ssems[j % 2])
        sh[NCH - 2].wait()
        sh[NCH - 1].wait()

    return k(table, flat_idx)


def _lstm_tile(m_ref, w_cat, bias_g, nrows):
    h0 = jnp.zeros((nrows, H), jnp.float32)
    c0 = jnp.zeros((nrows, H), jnp.float32)

    def step(t, hc):
        h, c = hc
        xt = m_ref[t].astype(jnp.bfloat16)
        xh = jnp.concatenate([xt, h.astype(jnp.bfloat16)], axis=1)
        gates = jnp.dot(xh, w_cat, preferred_element_type=jnp.float32) + bias_g
        i = _sigmoid_half(gates[:, :H])
        f = _sigmoid_half(gates[:, H:2 * H])
        g = jnp.tanh(gates[:, 2 * H:3 * H])
        o = _sigmoid_half(gates[:, 3 * H:])
        c2 = f * c + i * g
        h2 = o * jnp.tanh(c2)
        return (h2, c2)

    h, _ = lax.fori_loop(0, DEG, step, (h0, c0))
    return h


B1 = 1000


def _layer1_call(xin, mg, w_cat, bias_g, w_sn, bias_o):
    nt = N // B1

    def body(x_ref, m_ref, wc_ref, bg_ref, wsn_ref, bo_ref, o_ref):
        h = _lstm_tile(m_ref, wc_ref[...], bg_ref[...], B1)
        xh = jnp.concatenate([x_ref[...], h], axis=1)
        o_ref[...] = (jnp.dot(xh, wsn_ref[...],
                              preferred_element_type=jnp.float32)
                      + bo_ref[...])

    return pl.pallas_call(
        body,
        grid=(nt,),
        in_specs=[
            pl.BlockSpec((B1, D), lambda i: (i, 0)),
            pl.BlockSpec((DEG, B1, D), lambda i: (0, i, 0)),
            pl.BlockSpec((2 * D, 4 * H), lambda i: (0, 0)),
            pl.BlockSpec((1, 4 * H), lambda i: (0, 0)),
            pl.BlockSpec((2 * D, H), lambda i: (0, 0)),
            pl.BlockSpec((1, H), lambda i: (0, 0)),
        ],
        out_specs=pl.BlockSpec((B1, H), lambda i: (i, 0)),
        out_shape=jax.ShapeDtypeStruct((N, H), jnp.float32),
    )(xin, mg, w_cat, bias_g, w_sn, bias_o)


def _layer2_head_call(xin, mg, w_cat, bias_g, w_sn, bias_o,
                      fc1_W, fc1_b, fc3_W, fc3_b):
    nt = N // B1

    def body(x_ref, m_ref, wc_ref, bg_ref, wsn_ref, bo_ref,
             fc1w_ref, fc1b_ref, fc3w_ref, fc3b_ref, o_ref, mx_ref):
        i = pl.program_id(0)
        h = _lstm_tile(m_ref, wc_ref[...], bg_ref[...], B1)
        xh = jnp.concatenate([x_ref[...], h], axis=1)
        h2 = (jnp.dot(xh, wsn_ref[...], preferred_element_type=jnp.float32)
              + bo_ref[...])
        tmax = jnp.max(h2, axis=0, keepdims=True)

        @pl.when(i == 0)
        def _():
            mx_ref[...] = tmax

        @pl.when(i > 0)
        def _():
            mx_ref[...] = jnp.maximum(mx_ref[...], tmax)

        @pl.when(i == nt - 1)
        def _():
            g = mx_ref[...]
            z = jnp.maximum(
                jnp.dot(g, fc1w_ref[...], preferred_element_type=jnp.float32)
                + fc1b_ref[...], 0.0)
            o_ref[...] = (jnp.dot(z, fc3w_ref[...],
                                  preferred_element_type=jnp.float32)
                          + fc3b_ref[...])

    return pl.pallas_call(
        body,
        grid=(nt,),
        in_specs=[
            pl.BlockSpec((B1, H), lambda i: (i, 0)),
            pl.BlockSpec((DEG, B1, D), lambda i: (0, i, 0)),
            pl.BlockSpec((2 * H, 4 * H), lambda i: (0, 0)),
            pl.BlockSpec((1, 4 * H), lambda i: (0, 0)),
            pl.BlockSpec((2 * H, H), lambda i: (0, 0)),
            pl.BlockSpec((1, H), lambda i: (0, 0)),
            pl.BlockSpec((H, H // 2), lambda i: (0, 0)),
            pl.BlockSpec((1, H // 2), lambda i: (0, 0)),
            pl.BlockSpec((H // 2, 1), lambda i: (0, 0)),
            pl.BlockSpec((1, 1), lambda i: (0, 0)),
        ],
        out_specs=pl.BlockSpec((1, 1), lambda i: (0, 0)),
        out_shape=jax.ShapeDtypeStruct((1, 1), jnp.float32),
        scratch_shapes=[pltpu.VMEM((1, H), jnp.float32)],
    )(xin, mg, w_cat, bias_g, w_sn, bias_o, fc1_W, fc1_b, fc3_W, fc3_b)


def kernel(x, neighbor_idx, W_self1, W_neigh1, b1, Wih1, Whh1, bih1, bhh1,
           W_self2, W_neigh2, b2, Wih2, Whh2, bih2, bhh2,
           fc1_W, fc1_b, fc3_W, fc3_b):
    bf = jnp.bfloat16
    flat_idx = neighbor_idx.astype(jnp.int32).T.reshape(-1)

    wc1 = _fold_half(jnp.concatenate([Wih1, Whh1], axis=0)).astype(bf)
    wc2 = _fold_half(jnp.concatenate([Wih2, Whh2], axis=0)).astype(bf)
    wsn1 = jnp.concatenate([W_self1, W_neigh1], axis=0)
    wsn2 = jnp.concatenate([W_self2, W_neigh2], axis=0)
    bg1 = _fold_half((bih1 + bhh1).reshape(1, 4 * H))
    bg2 = _fold_half((bih2 + bhh2).reshape(1, 4 * H))
    bo1 = b1.reshape(1, H)
    bo2 = b2.reshape(1, H)

    mg1 = _sc_gather(x, flat_idx).reshape(DEG, N, D)
    h1 = _layer1_call(x, mg1, wc1, bg1, wsn1, bo1)

    mg2 = _sc_gather(h1, flat_idx).reshape(DEG, N, H)
    out = _layer2_head_call(h1, mg2, wc2, bg2, wsn2, bo2,
                            fc1_W, fc1_b.reshape(1, H // 2),
                            fc3_W, fc3_b.reshape(1, 1))
    return out
